# baseline (speedup 1.0000x reference)
import jax
import jax.numpy as jnp
from jax.experimental import pallas as pl
from jax.experimental.pallas import tpu as pltpu


def _enc(p_ref, wk_ref, bk_ref, wv_ref, bv_ref, k_ref, v_ref):
    p = p_ref[...]
    z = jnp.zeros((64, 64), jnp.float32)
    wkt = wk_ref[...].T
    wvt = wv_ref[...].T
    wkd = jnp.concatenate([jnp.concatenate([wkt, z], 1), jnp.concatenate([z, wkt], 1)], 0)
    wvd = jnp.concatenate([jnp.concatenate([wvt, z], 1), jnp.concatenate([z, wvt], 1)], 0)
    bkd = jnp.concatenate([bk_ref[...], bk_ref[...]], 1)
    bvd = jnp.concatenate([bv_ref[...], bv_ref[...]], 1)
    k_ref[...] = jnp.dot(p, wkd, preferred_element_type=jnp.float32) + bkd
    v_ref[...] = jnp.dot(p, wvd, preferred_element_type=jnp.float32) + bvd


def kernel(x, labels, prototype_vectors, Wk, bk, Wv, bv):
    p2 = prototype_vectors.reshape(4096, 128)
    BLK = 512
    k2, v2 = pl.pallas_call(
        _enc,
        grid=(4096 // BLK,),
        in_specs=[
            pl.BlockSpec((BLK, 128), lambda i: (i, 0)),
            pl.BlockSpec((64, 64), lambda i: (0, 0)),
            pl.BlockSpec((1, 64), lambda i: (0, 0)),
            pl.BlockSpec((64, 64), lambda i: (0, 0)),
            pl.BlockSpec((1, 64), lambda i: (0, 0)),
        ],
        out_specs=[pl.BlockSpec((BLK, 128), lambda i: (i, 0)),
                   pl.BlockSpec((BLK, 128), lambda i: (i, 0))],
        out_shape=[jax.ShapeDtypeStruct((4096, 128), jnp.float32),
                   jax.ShapeDtypeStruct((4096, 128), jnp.float32)],
        compiler_params=pltpu.CompilerParams(dimension_semantics=("parallel",)),
    )(p2, Wk, bk.reshape(1, 64), Wv, bv.reshape(1, 64))
    return (k2.reshape(8192, 64), v2.reshape(8192, 64))


# speedup vs baseline: 1.0725x; 1.0725x over previous
import jax
import jax.numpy as jnp
from jax.experimental import pallas as pl
from jax.experimental.pallas import tpu as pltpu


def _enc(p_ref, wk_ref, bk_ref, wv_ref, bv_ref, k_ref, v_ref):
    p = p_ref[...]
    z = jnp.zeros((64, 64), jnp.float32)
    wkt = wk_ref[...].T
    wvt = wv_ref[...].T
    wkd = jnp.concatenate([jnp.concatenate([wkt, z], 1), jnp.concatenate([z, wkt], 1)], 0)
    wvd = jnp.concatenate([jnp.concatenate([wvt, z], 1), jnp.concatenate([z, wvt], 1)], 0)
    bkd = jnp.concatenate([bk_ref[...], bk_ref[...]], 1)
    bvd = jnp.concatenate([bv_ref[...], bv_ref[...]], 1)
    k_ref[...] = jnp.dot(p, wkd, preferred_element_type=jnp.float32) + bkd
    v_ref[...] = jnp.dot(p, wvd, preferred_element_type=jnp.float32) + bvd


def kernel(x, labels, prototype_vectors, Wk, bk, Wv, bv):
    vm = pltpu.MemorySpace.VMEM
    p2 = pltpu.with_memory_space_constraint(prototype_vectors.reshape(4096, 128), vm)
    wkc = pltpu.with_memory_space_constraint(Wk, vm)
    bkc = pltpu.with_memory_space_constraint(bk.reshape(1, 64), vm)
    wvc = pltpu.with_memory_space_constraint(Wv, vm)
    bvc = pltpu.with_memory_space_constraint(bv.reshape(1, 64), vm)
    k2, v2 = pl.pallas_call(
        _enc,
        in_specs=[pl.BlockSpec(memory_space=vm)] * 5,
        out_specs=[pl.BlockSpec(memory_space=vm), pl.BlockSpec(memory_space=vm)],
        out_shape=[jax.ShapeDtypeStruct((4096, 128), jnp.float32),
                   jax.ShapeDtypeStruct((4096, 128), jnp.float32)],
    )(p2, wkc, bkc, wvc, bvc)
    return (k2.reshape(8192, 64), v2.reshape(8192, 64))


# speedup vs baseline: 1.1616x; 1.0831x over previous
import jax
import jax.numpy as jnp
from jax.experimental import pallas as pl


def _enc(p_ref, wk_ref, bk_ref, wv_ref, bv_ref, k_ref, v_ref):
    p = p_ref[...]
    z = jnp.zeros((64, 64), jnp.float32)
    wkt = wk_ref[...].T
    wvt = wv_ref[...].T
    wkd = jnp.concatenate([jnp.concatenate([wkt, z], 1), jnp.concatenate([z, wkt], 1)], 0)
    wvd = jnp.concatenate([jnp.concatenate([wvt, z], 1), jnp.concatenate([z, wvt], 1)], 0)
    bkd = jnp.concatenate([bk_ref[...], bk_ref[...]], 1)
    bvd = jnp.concatenate([bv_ref[...], bv_ref[...]], 1)
    k_ref[...] = jnp.dot(p, wkd, preferred_element_type=jnp.float32) + bkd
    v_ref[...] = jnp.dot(p, wvd, preferred_element_type=jnp.float32) + bvd


def kernel(x, labels, prototype_vectors, Wk, bk, Wv, bv):
    p2 = prototype_vectors.reshape(4096, 128)
    k2, v2 = pl.pallas_call(
        _enc,
        out_shape=[jax.ShapeDtypeStruct((4096, 128), jnp.float32),
                   jax.ShapeDtypeStruct((4096, 128), jnp.float32)],
    )(p2, Wk, bk.reshape(1, 64), Wv, bv.reshape(1, 64))
    return (k2.reshape(8192, 64), v2.reshape(8192, 64))


# speedup vs baseline: 1.3182x; 1.1348x over previous
import jax
import jax.numpy as jnp
from jax.experimental import pallas as pl


def _encode(p_ref, wk_ref, bk_ref, wv_ref, bv_ref, k_ref, v_ref):
    p = p_ref[...]
    k_ref[...] = (
        jnp.dot(p, wk_ref[...].T, preferred_element_type=jnp.float32) + bk_ref[...]
    )
    v_ref[...] = (
        jnp.dot(p, wv_ref[...].T, preferred_element_type=jnp.float32) + bv_ref[...]
    )


def kernel(x, labels, prototype_vectors, Wk, bk, Wv, bv):
    n, d = prototype_vectors.shape
    a = Wk.shape[0]
    k, v = pl.pallas_call(
        _encode,
        out_shape=[
            jax.ShapeDtypeStruct((n, a), jnp.float32),
            jax.ShapeDtypeStruct((n, a), jnp.float32),
        ],
    )(prototype_vectors, Wk, bk.reshape(1, a), Wv, bv.reshape(1, a))
    return (k, v)
